# baseline (speedup 1.0000x reference)
import jax
import jax.numpy as jnp
from jax.experimental import pallas as pl

B = 64
LEN = 256
D = 64
N = B * LEN


def _sage(x, ei, Wl, bl, Wr):
    src = ei[0]
    dst = ei[1]
    n = x.shape[0]
    agg = jax.ops.segment_sum(x[src], dst, num_segments=n)
    cnt = jax.ops.segment_sum(jnp.ones((src.shape[0],), x.dtype), dst, num_segments=n)
    mean = agg / jnp.clip(cnt, 1.0)[:, None]
    return mean @ Wl.T + bl + x @ Wr.T


def _segment_softmax(e, seg, nseg):
    m = jax.ops.segment_max(e, seg, num_segments=nseg)
    ex = jnp.exp(e - m[seg])
    s = jax.ops.segment_sum(ex, seg, num_segments=nseg)
    return ex / s[seg]


def _set2set(x, seg, nseg, Wih, Whh, bih, bhh):
    d = x.shape[1]
    q_star = jnp.zeros((nseg, 2 * d), x.dtype)
    h = jnp.zeros((nseg, d), x.dtype)
    c = jnp.zeros((nseg, d), x.dtype)
    for _ in range(2):
        gates = q_star @ Wih.T + bih + h @ Whh.T + bhh
        i = jax.nn.sigmoid(gates[:, :d])
        f = jax.nn.sigmoid(gates[:, d:2 * d])
        g = jnp.tanh(gates[:, 2 * d:3 * d])
        o = jax.nn.sigmoid(gates[:, 3 * d:])
        c = f * c + i * g
        h = o * jnp.tanh(c)
        q = h
        e = jnp.sum(x * q[seg], axis=-1)
        a = _segment_softmax(e, seg, nseg)
        r = jax.ops.segment_sum(a[:, None] * x, seg, num_segments=nseg)
        q_star = jnp.concatenate([q, r], axis=1)
    return q_star


def _touch(x):
    def body(x_ref, o_ref):
        o_ref[...] = x_ref[...]
    return pl.pallas_call(body, out_shape=jax.ShapeDtypeStruct(x.shape, x.dtype))(x)


def kernel(solute_data_zero, solute_data_one, solute_data_two, solute_data_three, solute_data_four, solvent_data_zero, solvent_data_one, solvent_data_two, solvent_data_three, solvent_data_four, solute_to_embedding, smile_zero, smile_one, smile_two, smile_three, smile_four, solute_adj, solvent_adj_zero, solvent_adj_one, solvent_adj_two, solvent_adj_three, solvent_adj_four, fc1_W, fc1_b, solute_c1_Wl, solute_c1_bl, solute_c1_Wr, solute_c2_Wl, solute_c2_bl, solute_c2_Wr, solvent_c1_Wl, solvent_c1_bl, solvent_c1_Wr, solvent_c2_Wl, solvent_c2_bl, solvent_c2_Wr, gru_Wih_f, gru_bih_f, gru_bhh_f, gru_Wih_b, gru_bih_b, gru_bhh_b, s2s_Wih, s2s_Whh, s2s_bih, s2s_bhh):
    seg = jnp.arange(N, dtype=jnp.int32) // LEN
    c1s = (solute_c1_Wl, solute_c1_bl, solute_c1_Wr)
    c2s = (solute_c2_Wl, solute_c2_bl, solute_c2_Wr)
    c1v = (solvent_c1_Wl, solvent_c1_bl, solvent_c1_Wr)
    c2v = (solvent_c2_Wl, solvent_c2_bl, solvent_c2_Wr)
    s2s = (s2s_Wih, s2s_Whh, s2s_bih, s2s_bhh)

    def branch(data, adj, c1, c2):
        xd = data.reshape(-1, data.shape[-1])
        init = xd @ fc1_W.T + fc1_b
        h1 = jax.nn.relu(_sage(init, adj, c1[0], c1[1], c1[2]))
        return _sage(h1, adj, c2[0], c2[1], c2[2]) + init

    solute_data = [solute_data_zero, solute_data_one, solute_data_two, solute_data_three, solute_data_four]
    solvent_data = [solvent_data_zero, solvent_data_one, solvent_data_two, solvent_data_three, solvent_data_four]
    solvent_adjs = [solvent_adj_zero, solvent_adj_one, solvent_adj_two, solvent_adj_three, solvent_adj_four]
    sols = []
    for i in range(5):
        su = branch(solute_data[i], solute_adj, c1s, c2s)
        sv = branch(solvent_data[i], solvent_adjs[i], c1v, c2v)
        su_p = _set2set(su, seg, B, s2s[0], s2s[1], s2s[2], s2s[3])
        sv_p = _set2set(sv, seg, B, s2s[0], s2s[1], s2s[2], s2s[3])
        sols.append(jnp.concatenate([su_p, sv_p], axis=1))
    out = jnp.concatenate(sols, axis=0)
    return _touch(out)


# speedup vs baseline: 7.6859x; 7.6859x over previous
import functools

import jax
import jax.numpy as jnp
from jax import lax
from jax.experimental import pallas as pl
from jax.experimental.pallas import tpu as pltpu
from jax.experimental.pallas import tpu_sc as plsc

B = 64
LEN = 256
NFEAT = 128
D = 64
E = 262144
N = B * LEN

NC = 2
NS = 16
EPC = E // NC
EPT = EPC // NS
CHUNK = 256
NCHUNK = EPT // CHUNK
RPT = N // NS
ZROWS = 128
ZROWS16 = 256

_mesh = plsc.VectorSubcoreMesh(core_axis_name="c", subcore_axis_name="s")


def _seg_job(x_hbm, adj_hbm, agg_hbm, j, cid, sid, acc, zbuf64, sidx, didx, rows, sem):
    for z in range(RPT // ZROWS):
        pltpu.sync_copy(zbuf64, acc.at[pl.ds(sid * RPT + z * ZROWS, ZROWS), :])
    plsc.subcore_barrier()
    base0 = cid * EPC + sid * EPT

    def chunk(i, carry):
        b = base0 + i * CHUNK
        pltpu.sync_copy(adj_hbm.at[0, pl.ds(b, CHUNK)], sidx)
        pltpu.sync_copy(adj_hbm.at[1, pl.ds(b, CHUNK)], didx)
        pltpu.async_copy(x_hbm.at[sidx], rows, sem).wait()
        pltpu.sync_copy(rows, acc.at[didx], add=True)
        return carry

    lax.fori_loop(0, NCHUNK, chunk, 0)
    plsc.subcore_barrier()
    for c in range(NC):
        @pl.when(cid == c)
        def _():
            pltpu.sync_copy(acc.at[pl.ds(sid * RPT, RPT), :],
                            agg_hbm.at[c, j, pl.ds(sid * RPT, RPT), :])
    plsc.subcore_barrier()


def _cnt_job(adj_hbm, cnt_hbm, a, cid, sid, cacc, zbuf16, obuf16, didx, sem):
    for z in range(RPT // ZROWS16):
        pltpu.sync_copy(zbuf16, cacc.at[pl.ds(sid * RPT + z * ZROWS16, ZROWS16), :])
    plsc.subcore_barrier()
    base0 = cid * EPC + sid * EPT

    def chunk(i, carry):
        b = base0 + i * CHUNK
        pltpu.sync_copy(adj_hbm.at[1, pl.ds(b, CHUNK)], didx)
        pltpu.sync_copy(obuf16, cacc.at[didx], add=True)
        return carry

    lax.fori_loop(0, NCHUNK, chunk, 0)
    plsc.subcore_barrier()
    for c in range(NC):
        @pl.when(cid == c)
        def _():
            pltpu.sync_copy(cacc.at[pl.ds(sid * RPT, RPT), :],
                            cnt_hbm.at[c, a, pl.ds(sid * RPT, RPT), :])
    plsc.subcore_barrier()


def _make_sc_seg(with_counts):
    out_type = [jax.ShapeDtypeStruct((NC, 10, N, D), jnp.float32)]
    if with_counts:
        out_type.append(jax.ShapeDtypeStruct((NC, 6, N, 16), jnp.float32))

    @functools.partial(
        pl.kernel,
        out_type=tuple(out_type) if with_counts else out_type[0],
        mesh=_mesh,
        compiler_params=pltpu.CompilerParams(use_tc_tiling_on_sc=False),
        scratch_types=[
            pltpu.VMEM_SHARED((N, D), jnp.float32),
            pltpu.VMEM_SHARED((N, 16), jnp.float32),
            pltpu.VMEM((ZROWS, D), jnp.float32),
            pltpu.VMEM((ZROWS16, 16), jnp.float32),
            pltpu.VMEM((CHUNK, 16), jnp.float32),
            pltpu.VMEM((CHUNK,), jnp.int32),
            pltpu.VMEM((CHUNK,), jnp.int32),
            pltpu.VMEM((CHUNK, D), jnp.float32),
            pltpu.SemaphoreType.DMA,
        ],
    )
    def sc_seg(xs, a0, a1, a2, a3, a4, a5, zeros64, zeros16, ones16, *out_and_scratch):
        if with_counts:
            agg, cnt, acc, cacc, zbuf64, zbuf16, obuf16, sidx, didx, rows, sem = out_and_scratch
        else:
            agg, acc, cacc, zbuf64, zbuf16, obuf16, sidx, didx, rows, sem = out_and_scratch
        cid = lax.axis_index("c")
        sid = lax.axis_index("s")
        adjs = [a0, a1, a2, a3, a4, a5]
        pltpu.sync_copy(zeros64, zbuf64)
        if with_counts:
            pltpu.sync_copy(zeros16, zbuf16)
            pltpu.sync_copy(ones16, obuf16)
            for a in range(6):
                _cnt_job(adjs[a], cnt, a, cid, sid, cacc, zbuf16, obuf16, didx, sem)
        for j in range(10):
            adj = adjs[0] if j < 5 else adjs[j - 4]
            _seg_job(xs.at[j], adj, agg, j, cid, sid, acc, zbuf64, sidx, didx, rows, sem)

    return sc_seg


_sc_seg_counts = _make_sc_seg(True)
_sc_seg_plain = _make_sc_seg(False)


_RB = 512
_NG = N // _RB


def _fc1_body(*refs):
    xs = refs[:10]
    w, b = refs[10], refs[11]
    o = refs[12]
    wv = w[...]
    bv = b[...]
    for j in range(10):
        o[j] = jnp.dot(xs[j][...], wv, preferred_element_type=jnp.float32) + bv


def _fc1_all(datas, wT, b2):
    return pl.pallas_call(
        _fc1_body,
        grid=(_NG,),
        in_specs=[pl.BlockSpec((_RB, NFEAT), lambda r: (r, 0))] * 10
        + [pl.BlockSpec((NFEAT, D), lambda r: (0, 0)),
           pl.BlockSpec((1, D), lambda r: (0, 0))],
        out_specs=pl.BlockSpec((10, _RB, D), lambda r: (0, r, 0)),
        out_shape=jax.ShapeDtypeStruct((10, N, D), jnp.float32),
    )(*datas, wT, b2)


def _sage_body(relu, residual, agg_ref, cnt_ref, x_ref, res_ref,
               wlT_u, bl_u, wrT_u, wlT_v, bl_v, wrT_v, o_ref):
    cnt = cnt_ref[0, :, :, 0:1] + cnt_ref[1, :, :, 0:1]
    rcnt = 1.0 / jnp.maximum(cnt, 1.0)
    for j in range(10):
        a = 0 if j < 5 else j - 4
        wlT = wlT_u[...] if j < 5 else wlT_v[...]
        bl = bl_u[...] if j < 5 else bl_v[...]
        wrT = wrT_u[...] if j < 5 else wrT_v[...]
        mean = (agg_ref[0, j] + agg_ref[1, j]) * rcnt[a]
        h = (jnp.dot(mean, wlT, preferred_element_type=jnp.float32) + bl
             + jnp.dot(x_ref[j], wrT, preferred_element_type=jnp.float32))
        if relu:
            h = jnp.maximum(h, 0.0)
        if residual:
            h = h + res_ref[j]
        o_ref[j] = h


def _sage_all(relu, residual, agg, cnt, x, res, wlT_u, bl_u, wrT_u, wlT_v, bl_v, wrT_v):
    body = functools.partial(_sage_body, relu, residual)
    return pl.pallas_call(
        body,
        grid=(_NG,),
        in_specs=[
            pl.BlockSpec((NC, 10, _RB, D), lambda r: (0, 0, r, 0)),
            pl.BlockSpec((NC, 6, _RB, 16), lambda r: (0, 0, r, 0)),
            pl.BlockSpec((10, _RB, D), lambda r: (0, r, 0)),
            pl.BlockSpec((10, _RB, D), lambda r: (0, r, 0)),
            pl.BlockSpec((D, D), lambda r: (0, 0)),
            pl.BlockSpec((1, D), lambda r: (0, 0)),
            pl.BlockSpec((D, D), lambda r: (0, 0)),
            pl.BlockSpec((D, D), lambda r: (0, 0)),
            pl.BlockSpec((1, D), lambda r: (0, 0)),
            pl.BlockSpec((D, D), lambda r: (0, 0)),
        ],
        out_specs=pl.BlockSpec((10, _RB, D), lambda r: (0, r, 0)),
        out_shape=jax.ShapeDtypeStruct((10, N, D), jnp.float32),
    )(agg, cnt, x, res, wlT_u, bl_u, wrT_u, wlT_v, bl_v, wrT_v)


def _s2s_body(x_ref, wihT_ref, whhT_ref, bih_ref, bhh_ref, o_ref):
    x = x_ref[0]
    wihT = wihT_ref[...]
    whhT = whhT_ref[...]
    bih = bih_ref[...]
    bhh = bhh_ref[...]
    q_star = jnp.zeros((B, 2 * D), jnp.float32)
    h = jnp.zeros((B, D), jnp.float32)
    c = jnp.zeros((B, D), jnp.float32)
    for _ in range(2):
        gates = (jnp.dot(q_star, wihT, preferred_element_type=jnp.float32) + bih
                 + jnp.dot(h, whhT, preferred_element_type=jnp.float32) + bhh)
        ig = jax.nn.sigmoid(gates[:, :D])
        fg = jax.nn.sigmoid(gates[:, D:2 * D])
        gg = jnp.tanh(gates[:, 2 * D:3 * D])
        og = jax.nn.sigmoid(gates[:, 3 * D:])
        c = fg * c + ig * gg
        h = og * jnp.tanh(c)
        e = jnp.sum(x * h[:, None, :], axis=-1)
        m = jnp.max(e, axis=1, keepdims=True)
        ex = jnp.exp(e - m)
        s = jnp.sum(ex, axis=1, keepdims=True)
        a = ex / s
        r = jnp.sum(x * a[:, :, None], axis=1)
        q_star = jnp.concatenate([h, r], axis=1)
    o_ref[0] = q_star


def _s2s_all(x4, wihT, whhT, bih2, bhh2):
    return pl.pallas_call(
        _s2s_body,
        grid=(10,),
        in_specs=[
            pl.BlockSpec((1, B, LEN, D), lambda j: (j, 0, 0, 0)),
            pl.BlockSpec((2 * D, 4 * D), lambda j: (0, 0)),
            pl.BlockSpec((D, 4 * D), lambda j: (0, 0)),
            pl.BlockSpec((1, 4 * D), lambda j: (0, 0)),
            pl.BlockSpec((1, 4 * D), lambda j: (0, 0)),
        ],
        out_specs=pl.BlockSpec((1, B, 2 * D), lambda j: (j, 0, 0)),
        out_shape=jax.ShapeDtypeStruct((10, B, 2 * D), jnp.float32),
    )(x4, wihT, whhT, bih2, bhh2)


def kernel(solute_data_zero, solute_data_one, solute_data_two, solute_data_three, solute_data_four, solvent_data_zero, solvent_data_one, solvent_data_two, solvent_data_three, solvent_data_four, solute_to_embedding, smile_zero, smile_one, smile_two, smile_three, smile_four, solute_adj, solvent_adj_zero, solvent_adj_one, solvent_adj_two, solvent_adj_three, solvent_adj_four, fc1_W, fc1_b, solute_c1_Wl, solute_c1_bl, solute_c1_Wr, solute_c2_Wl, solute_c2_bl, solute_c2_Wr, solvent_c1_Wl, solvent_c1_bl, solvent_c1_Wr, solvent_c2_Wl, solvent_c2_bl, solvent_c2_Wr, gru_Wih_f, gru_bih_f, gru_bhh_f, gru_Wih_b, gru_bih_b, gru_bhh_b, s2s_Wih, s2s_Whh, s2s_bih, s2s_bhh):
    datas = [d.reshape(N, NFEAT) for d in
             (solute_data_zero, solute_data_one, solute_data_two, solute_data_three, solute_data_four,
              solvent_data_zero, solvent_data_one, solvent_data_two, solvent_data_three, solvent_data_four)]
    adjs = (solute_adj, solvent_adj_zero, solvent_adj_one, solvent_adj_two,
            solvent_adj_three, solvent_adj_four)
    zeros64 = jnp.zeros((ZROWS, D), jnp.float32)
    zeros16 = jnp.zeros((ZROWS16, 16), jnp.float32)
    ones16 = jnp.ones((CHUNK, 16), jnp.float32)

    x0 = _fc1_all(datas, fc1_W.T, fc1_b.reshape(1, D))

    agg1, cnt = _sc_seg_counts(x0, *adjs, zeros64, zeros16, ones16)

    x1 = _sage_all(True, False, agg1, cnt, x0, x0,
                   solute_c1_Wl.T, solute_c1_bl.reshape(1, D), solute_c1_Wr.T,
                   solvent_c1_Wl.T, solvent_c1_bl.reshape(1, D), solvent_c1_Wr.T)

    agg2 = _sc_seg_plain(x1, *adjs, zeros64, zeros16, ones16)

    x2 = _sage_all(False, True, agg2, cnt, x1, x0,
                   solute_c2_Wl.T, solute_c2_bl.reshape(1, D), solute_c2_Wr.T,
                   solvent_c2_Wl.T, solvent_c2_bl.reshape(1, D), solvent_c2_Wr.T)

    x4 = x2.reshape(10, B, LEN, D)
    pooled = _s2s_all(x4, s2s_Wih.T, s2s_Whh.T,
                      s2s_bih.reshape(1, 4 * D), s2s_bhh.reshape(1, 4 * D))

    out = jnp.concatenate([pooled[:5], pooled[5:]], axis=2)
    return out.reshape(5 * B, 4 * D)


# speedup vs baseline: 10.0317x; 1.3052x over previous
import functools

import jax
import jax.numpy as jnp
from jax import lax
from jax.experimental import pallas as pl
from jax.experimental.pallas import tpu as pltpu
from jax.experimental.pallas import tpu_sc as plsc

B = 64
LEN = 256
NFEAT = 128
D = 64
E = 262144
N = B * LEN

NC = 2
NS = 16
EPC = E // NC
EPT = EPC // NS
CHUNK = 256
NCHUNK = EPT // CHUNK
RPT = N // NS
ZROWS = 128
ZROWS16 = 128

_mesh = plsc.VectorSubcoreMesh(core_axis_name="c", subcore_axis_name="s")


def _seg_job(x_hbm, adj_hbm, agg_hbm, j, cid, sid, acc, zbuf64, sidx, didx, rows, sems):
    for z in range(RPT // ZROWS):
        pltpu.sync_copy(zbuf64, acc.at[pl.ds(sid * RPT + z * ZROWS, ZROWS), :])
    plsc.subcore_barrier()
    base0 = cid * EPC + sid * EPT

    def issue(k, b):
        off = base0 + k * CHUNK
        pltpu.sync_copy(adj_hbm.at[0, pl.ds(off, CHUNK)], sidx[b])
        pltpu.sync_copy(adj_hbm.at[1, pl.ds(off, CHUNK)], didx[b])
        pltpu.async_copy(x_hbm.at[sidx[b]], rows[b], sems[b])

    for b in range(2):
        issue(b, b)

    def outer(i, carry):
        for b in range(2):
            k = 2 * i + b
            pltpu.make_async_copy(x_hbm.at[sidx[b]], rows[b], sems[b]).wait()
            pltpu.sync_copy(rows[b], acc.at[didx[b]], add=True)

            @pl.when(k + 2 < NCHUNK)
            def _():
                issue(k + 2, b)
        return carry

    lax.fori_loop(0, NCHUNK // 2, outer, 0)
    plsc.subcore_barrier()
    for c in range(NC):
        @pl.when(cid == c)
        def _():
            pltpu.sync_copy(acc.at[pl.ds(sid * RPT, RPT), :],
                            agg_hbm.at[c, j, pl.ds(sid * RPT, RPT), :])
    plsc.subcore_barrier()


def _cnt_job(adj_hbm, cnt_hbm, a, cid, sid, cacc, zbuf16, obuf16, didx, sem):
    for z in range(RPT // ZROWS16):
        pltpu.sync_copy(zbuf16, cacc.at[pl.ds(sid * RPT + z * ZROWS16, ZROWS16), :])
    plsc.subcore_barrier()
    base0 = cid * EPC + sid * EPT

    def chunk(i, carry):
        b = base0 + i * CHUNK
        pltpu.sync_copy(adj_hbm.at[1, pl.ds(b, CHUNK)], didx)
        pltpu.sync_copy(obuf16, cacc.at[didx], add=True)
        return carry

    lax.fori_loop(0, NCHUNK, chunk, 0)
    plsc.subcore_barrier()
    for c in range(NC):
        @pl.when(cid == c)
        def _():
            pltpu.sync_copy(cacc.at[pl.ds(sid * RPT, RPT), :],
                            cnt_hbm.at[c, a, pl.ds(sid * RPT, RPT), :])
    plsc.subcore_barrier()


def _make_sc_seg(with_counts):
    out_type = [jax.ShapeDtypeStruct((NC, 10, N, D), jnp.float32)]
    if with_counts:
        out_type.append(jax.ShapeDtypeStruct((NC, 6, N, 16), jnp.float32))

    @functools.partial(
        pl.kernel,
        out_type=tuple(out_type) if with_counts else out_type[0],
        mesh=_mesh,
        compiler_params=pltpu.CompilerParams(use_tc_tiling_on_sc=False),
        scratch_types=[
            pltpu.VMEM_SHARED((N, D), jnp.float32),
            pltpu.VMEM_SHARED((N, 16), jnp.float32),
            pltpu.VMEM((ZROWS, D), jnp.float32),
            pltpu.VMEM((ZROWS16, 16), jnp.float32),
            pltpu.VMEM((CHUNK, 16), jnp.float32),
            pltpu.VMEM((CHUNK,), jnp.int32),
            pltpu.VMEM((CHUNK,), jnp.int32),
            pltpu.VMEM((CHUNK,), jnp.int32),
            pltpu.VMEM((CHUNK,), jnp.int32),
            pltpu.VMEM((CHUNK, D), jnp.float32),
            pltpu.VMEM((CHUNK, D), jnp.float32),
            pltpu.SemaphoreType.DMA,
            pltpu.SemaphoreType.DMA,
        ],
    )
    def sc_seg(xs, a0, a1, a2, a3, a4, a5, zeros64, zeros16, ones16, *out_and_scratch):
        if with_counts:
            (agg, cnt, acc, cacc, zbuf64, zbuf16, obuf16, sidx0, sidx1,
             didx0, didx1, rows0, rows1, sem0, sem1) = out_and_scratch
        else:
            (agg, acc, cacc, zbuf64, zbuf16, obuf16, sidx0, sidx1,
             didx0, didx1, rows0, rows1, sem0, sem1) = out_and_scratch
        sidx = (sidx0, sidx1)
        didx = (didx0, didx1)
        rows = (rows0, rows1)
        sems = (sem0, sem1)
        cid = lax.axis_index("c")
        sid = lax.axis_index("s")
        adjs = [a0, a1, a2, a3, a4, a5]
        pltpu.sync_copy(zeros64, zbuf64)
        if with_counts:
            pltpu.sync_copy(zeros16, zbuf16)
            pltpu.sync_copy(ones16, obuf16)
            for a in range(6):
                _cnt_job(adjs[a], cnt, a, cid, sid, cacc, zbuf16, obuf16, didx0, sem0)
        for j in range(10):
            adj = adjs[0] if j < 5 else adjs[j - 4]
            _seg_job(xs.at[j], adj, agg, j, cid, sid, acc, zbuf64, sidx, didx, rows, sems)

    return sc_seg


_sc_seg_counts = _make_sc_seg(True)
_sc_seg_plain = _make_sc_seg(False)


_RB = 512
_NG = N // _RB


def _fc1_body(*refs):
    xs = refs[:10]
    w, b = refs[10], refs[11]
    o = refs[12]
    wv = w[...]
    bv = b[...]
    for j in range(10):
        o[j] = jnp.dot(xs[j][...], wv, preferred_element_type=jnp.float32) + bv


def _fc1_all(datas, wT, b2):
    return pl.pallas_call(
        _fc1_body,
        grid=(_NG,),
        in_specs=[pl.BlockSpec((_RB, NFEAT), lambda r: (r, 0))] * 10
        + [pl.BlockSpec((NFEAT, D), lambda r: (0, 0)),
           pl.BlockSpec((1, D), lambda r: (0, 0))],
        out_specs=pl.BlockSpec((10, _RB, D), lambda r: (0, r, 0)),
        out_shape=jax.ShapeDtypeStruct((10, N, D), jnp.float32),
    )(*datas, wT, b2)


def _sage_body(relu, residual, agg_ref, cnt_ref, x_ref, res_ref,
               wlT_u, bl_u, wrT_u, wlT_v, bl_v, wrT_v, o_ref):
    cnt = cnt_ref[0, :, :, 0:1] + cnt_ref[1, :, :, 0:1]
    rcnt = 1.0 / jnp.maximum(cnt, 1.0)
    for j in range(10):
        a = 0 if j < 5 else j - 4
        wlT = wlT_u[...] if j < 5 else wlT_v[...]
        bl = bl_u[...] if j < 5 else bl_v[...]
        wrT = wrT_u[...] if j < 5 else wrT_v[...]
        mean = (agg_ref[0, j] + agg_ref[1, j]) * rcnt[a]
        h = (jnp.dot(mean, wlT, preferred_element_type=jnp.float32) + bl
             + jnp.dot(x_ref[j], wrT, preferred_element_type=jnp.float32))
        if relu:
            h = jnp.maximum(h, 0.0)
        if residual:
            h = h + res_ref[j]
        o_ref[j] = h


def _sage_all(relu, residual, agg, cnt, x, res, wlT_u, bl_u, wrT_u, wlT_v, bl_v, wrT_v):
    body = functools.partial(_sage_body, relu, residual)
    return pl.pallas_call(
        body,
        grid=(_NG,),
        in_specs=[
            pl.BlockSpec((NC, 10, _RB, D), lambda r: (0, 0, r, 0)),
            pl.BlockSpec((NC, 6, _RB, 16), lambda r: (0, 0, r, 0)),
            pl.BlockSpec((10, _RB, D), lambda r: (0, r, 0)),
            pl.BlockSpec((10, _RB, D), lambda r: (0, r, 0)),
            pl.BlockSpec((D, D), lambda r: (0, 0)),
            pl.BlockSpec((1, D), lambda r: (0, 0)),
            pl.BlockSpec((D, D), lambda r: (0, 0)),
            pl.BlockSpec((D, D), lambda r: (0, 0)),
            pl.BlockSpec((1, D), lambda r: (0, 0)),
            pl.BlockSpec((D, D), lambda r: (0, 0)),
        ],
        out_specs=pl.BlockSpec((10, _RB, D), lambda r: (0, r, 0)),
        out_shape=jax.ShapeDtypeStruct((10, N, D), jnp.float32),
    )(agg, cnt, x, res, wlT_u, bl_u, wrT_u, wlT_v, bl_v, wrT_v)


def _s2s_body(x_ref, wihT_ref, whhT_ref, bih_ref, bhh_ref, o_ref):
    x = x_ref[0]
    wihT = wihT_ref[...]
    whhT = whhT_ref[...]
    bih = bih_ref[...]
    bhh = bhh_ref[...]
    q_star = jnp.zeros((B, 2 * D), jnp.float32)
    h = jnp.zeros((B, D), jnp.float32)
    c = jnp.zeros((B, D), jnp.float32)
    for _ in range(2):
        gates = (jnp.dot(q_star, wihT, preferred_element_type=jnp.float32) + bih
                 + jnp.dot(h, whhT, preferred_element_type=jnp.float32) + bhh)
        ig = jax.nn.sigmoid(gates[:, :D])
        fg = jax.nn.sigmoid(gates[:, D:2 * D])
        gg = jnp.tanh(gates[:, 2 * D:3 * D])
        og = jax.nn.sigmoid(gates[:, 3 * D:])
        c = fg * c + ig * gg
        h = og * jnp.tanh(c)
        e = jnp.sum(x * h[:, None, :], axis=-1)
        m = jnp.max(e, axis=1, keepdims=True)
        ex = jnp.exp(e - m)
        s = jnp.sum(ex, axis=1, keepdims=True)
        a = ex / s
        r = jnp.sum(x * a[:, :, None], axis=1)
        q_star = jnp.concatenate([h, r], axis=1)
    o_ref[0] = q_star


def _s2s_all(x4, wihT, whhT, bih2, bhh2):
    return pl.pallas_call(
        _s2s_body,
        grid=(10,),
        in_specs=[
            pl.BlockSpec((1, B, LEN, D), lambda j: (j, 0, 0, 0)),
            pl.BlockSpec((2 * D, 4 * D), lambda j: (0, 0)),
            pl.BlockSpec((D, 4 * D), lambda j: (0, 0)),
            pl.BlockSpec((1, 4 * D), lambda j: (0, 0)),
            pl.BlockSpec((1, 4 * D), lambda j: (0, 0)),
        ],
        out_specs=pl.BlockSpec((1, B, 2 * D), lambda j: (j, 0, 0)),
        out_shape=jax.ShapeDtypeStruct((10, B, 2 * D), jnp.float32),
    )(x4, wihT, whhT, bih2, bhh2)


def kernel(solute_data_zero, solute_data_one, solute_data_two, solute_data_three, solute_data_four, solvent_data_zero, solvent_data_one, solvent_data_two, solvent_data_three, solvent_data_four, solute_to_embedding, smile_zero, smile_one, smile_two, smile_three, smile_four, solute_adj, solvent_adj_zero, solvent_adj_one, solvent_adj_two, solvent_adj_three, solvent_adj_four, fc1_W, fc1_b, solute_c1_Wl, solute_c1_bl, solute_c1_Wr, solute_c2_Wl, solute_c2_bl, solute_c2_Wr, solvent_c1_Wl, solvent_c1_bl, solvent_c1_Wr, solvent_c2_Wl, solvent_c2_bl, solvent_c2_Wr, gru_Wih_f, gru_bih_f, gru_bhh_f, gru_Wih_b, gru_bih_b, gru_bhh_b, s2s_Wih, s2s_Whh, s2s_bih, s2s_bhh):
    datas = [d.reshape(N, NFEAT) for d in
             (solute_data_zero, solute_data_one, solute_data_two, solute_data_three, solute_data_four,
              solvent_data_zero, solvent_data_one, solvent_data_two, solvent_data_three, solvent_data_four)]
    adjs = (solute_adj, solvent_adj_zero, solvent_adj_one, solvent_adj_two,
            solvent_adj_three, solvent_adj_four)
    zeros64 = jnp.zeros((ZROWS, D), jnp.float32)
    zeros16 = jnp.zeros((ZROWS16, 16), jnp.float32)
    ones16 = jnp.ones((CHUNK, 16), jnp.float32)

    x0 = _fc1_all(datas, fc1_W.T, fc1_b.reshape(1, D))

    agg1, cnt = _sc_seg_counts(x0, *adjs, zeros64, zeros16, ones16)

    x1 = _sage_all(True, False, agg1, cnt, x0, x0,
                   solute_c1_Wl.T, solute_c1_bl.reshape(1, D), solute_c1_Wr.T,
                   solvent_c1_Wl.T, solvent_c1_bl.reshape(1, D), solvent_c1_Wr.T)

    agg2 = _sc_seg_plain(x1, *adjs, zeros64, zeros16, ones16)

    x2 = _sage_all(False, True, agg2, cnt, x1, x0,
                   solute_c2_Wl.T, solute_c2_bl.reshape(1, D), solute_c2_Wr.T,
                   solvent_c2_Wl.T, solvent_c2_bl.reshape(1, D), solvent_c2_Wr.T)

    x4 = x2.reshape(10, B, LEN, D)
    pooled = _s2s_all(x4, s2s_Wih.T, s2s_Whh.T,
                      s2s_bih.reshape(1, 4 * D), s2s_bhh.reshape(1, 4 * D))

    out = jnp.concatenate([pooled[:5], pooled[5:]], axis=2)
    return out.reshape(5 * B, 4 * D)


# speedup vs baseline: 10.9438x; 1.0909x over previous
import functools

import jax
import jax.numpy as jnp
from jax import lax
from jax.experimental import pallas as pl
from jax.experimental.pallas import tpu as pltpu
from jax.experimental.pallas import tpu_sc as plsc

B = 64
LEN = 256
NFEAT = 128
D = 64
E = 262144
N = B * LEN

NC = 2
NS = 16
EPC = E // NC
EPT = EPC // NS
CHUNK = 128
NCHUNK = EPT // CHUNK
RPT = N // NS

_mesh = plsc.VectorSubcoreMesh(core_axis_name="c", subcore_axis_name="s")


def _seg_job(x_hbm, adj_hbm, agg_hbm, j, cid, sid, acc, zrows_hbm, sidxall, didxall, rows, sems):
    pltpu.sync_copy(zrows_hbm, acc.at[pl.ds(sid * RPT, RPT), :])
    cbase = (cid * EPC + sid * EPT) // CHUNK
    pltpu.sync_copy(adj_hbm.at[0, pl.ds(cbase, NCHUNK), :], sidxall)
    pltpu.sync_copy(adj_hbm.at[1, pl.ds(cbase, NCHUNK), :], didxall)
    plsc.subcore_barrier()

    def issue(k, b):
        pltpu.async_copy(x_hbm.at[sidxall.at[k]], rows[b], sems[b])

    for b in range(2):
        issue(b, b)

    def outer(i, carry):
        for b in range(2):
            k = 2 * i + b
            pltpu.make_async_copy(x_hbm.at[sidxall.at[k]], rows[b], sems[b]).wait()
            pltpu.sync_copy(rows[b], acc.at[didxall.at[k]], add=True)

            @pl.when(k + 2 < NCHUNK)
            def _():
                issue(k + 2, b)
        return carry

    lax.fori_loop(0, NCHUNK // 2, outer, 0)
    plsc.subcore_barrier()
    for c in range(NC):
        @pl.when(cid == c)
        def _():
            pltpu.sync_copy(acc.at[pl.ds(sid * RPT, RPT), :],
                            agg_hbm.at[c, j, pl.ds(sid * RPT, RPT), :])
    plsc.subcore_barrier()


def _cnt_job(adj_hbm, cnt_hbm, a, cid, sid, cacc, zrows16_hbm, obuf16, didxall, sem):
    pltpu.sync_copy(zrows16_hbm, cacc.at[pl.ds(sid * RPT, RPT), :])
    cbase = (cid * EPC + sid * EPT) // CHUNK
    pltpu.sync_copy(adj_hbm.at[1, pl.ds(cbase, NCHUNK), :], didxall)
    plsc.subcore_barrier()

    def chunk(i, carry):
        pltpu.sync_copy(obuf16, cacc.at[didxall.at[i]], add=True)
        return carry

    lax.fori_loop(0, NCHUNK, chunk, 0)
    plsc.subcore_barrier()
    for c in range(NC):
        @pl.when(cid == c)
        def _():
            pltpu.sync_copy(cacc.at[pl.ds(sid * RPT, RPT), :],
                            cnt_hbm.at[c, a, pl.ds(sid * RPT, RPT), :])
    plsc.subcore_barrier()


def _make_sc_seg(with_counts):
    out_type = [jax.ShapeDtypeStruct((NC, 10, N, D), jnp.float32)]
    if with_counts:
        out_type.append(jax.ShapeDtypeStruct((NC, 6, N, 16), jnp.float32))

    @functools.partial(
        pl.kernel,
        out_type=tuple(out_type) if with_counts else out_type[0],
        mesh=_mesh,
        compiler_params=pltpu.CompilerParams(use_tc_tiling_on_sc=False),
        scratch_types=[
            pltpu.VMEM_SHARED((N, D), jnp.float32),
            pltpu.VMEM_SHARED((N, 16), jnp.float32),
            pltpu.VMEM((CHUNK, 16), jnp.float32),
            pltpu.VMEM((NCHUNK, CHUNK), jnp.int32),
            pltpu.VMEM((NCHUNK, CHUNK), jnp.int32),
            pltpu.VMEM((CHUNK, D), jnp.float32),
            pltpu.VMEM((CHUNK, D), jnp.float32),
            pltpu.SemaphoreType.DMA,
            pltpu.SemaphoreType.DMA,
        ],
    )
    def sc_seg(xs, a0, a1, a2, a3, a4, a5, zeros64, zeros16, ones16, *out_and_scratch):
        if with_counts:
            (agg, cnt, acc, cacc, obuf16, sidxall, didxall,
             rows0, rows1, sem0, sem1) = out_and_scratch
        else:
            (agg, acc, cacc, obuf16, sidxall, didxall,
             rows0, rows1, sem0, sem1) = out_and_scratch
        rows = (rows0, rows1)
        sems = (sem0, sem1)
        cid = lax.axis_index("c")
        sid = lax.axis_index("s")
        adjs = [a0, a1, a2, a3, a4, a5]
        if with_counts:
            pltpu.sync_copy(ones16, obuf16)
            for a in range(6):
                _cnt_job(adjs[a], cnt, a, cid, sid, cacc, zeros16, obuf16, didxall, sem0)
        for j in range(10):
            adj = adjs[0] if j < 5 else adjs[j - 4]
            _seg_job(xs.at[j], adj, agg, j, cid, sid, acc, zeros64, sidxall, didxall, rows, sems)

    return sc_seg


_sc_seg_counts = _make_sc_seg(True)
_sc_seg_plain = _make_sc_seg(False)


_RB = 512
_NG = N // _RB


def _fc1_body(*refs):
    xs = refs[:10]
    w, b = refs[10], refs[11]
    o = refs[12]
    wv = w[...]
    bv = b[...]
    for j in range(10):
        o[j] = jnp.dot(xs[j][...], wv, preferred_element_type=jnp.float32) + bv


def _fc1_all(datas, wT, b2):
    return pl.pallas_call(
        _fc1_body,
        grid=(_NG,),
        in_specs=[pl.BlockSpec((_RB, NFEAT), lambda r: (r, 0))] * 10
        + [pl.BlockSpec((NFEAT, D), lambda r: (0, 0)),
           pl.BlockSpec((1, D), lambda r: (0, 0))],
        out_specs=pl.BlockSpec((10, _RB, D), lambda r: (0, r, 0)),
        out_shape=jax.ShapeDtypeStruct((10, N, D), jnp.float32),
    )(*datas, wT, b2)


def _sage_body(relu, residual, agg_ref, cnt_ref, x_ref, res_ref,
               wlT_u, bl_u, wrT_u, wlT_v, bl_v, wrT_v, o_ref):
    cnt = cnt_ref[0, :, :, 0:1] + cnt_ref[1, :, :, 0:1]
    rcnt = 1.0 / jnp.maximum(cnt, 1.0)
    for j in range(10):
        a = 0 if j < 5 else j - 4
        wlT = wlT_u[...] if j < 5 else wlT_v[...]
        bl = bl_u[...] if j < 5 else bl_v[...]
        wrT = wrT_u[...] if j < 5 else wrT_v[...]
        mean = (agg_ref[0, j] + agg_ref[1, j]) * rcnt[a]
        h = (jnp.dot(mean, wlT, preferred_element_type=jnp.float32) + bl
             + jnp.dot(x_ref[j], wrT, preferred_element_type=jnp.float32))
        if relu:
            h = jnp.maximum(h, 0.0)
        if residual:
            h = h + res_ref[j]
        o_ref[j] = h


def _sage_all(relu, residual, agg, cnt, x, res, wlT_u, bl_u, wrT_u, wlT_v, bl_v, wrT_v):
    body = functools.partial(_sage_body, relu, residual)
    return pl.pallas_call(
        body,
        grid=(_NG,),
        in_specs=[
            pl.BlockSpec((NC, 10, _RB, D), lambda r: (0, 0, r, 0)),
            pl.BlockSpec((NC, 6, _RB, 16), lambda r: (0, 0, r, 0)),
            pl.BlockSpec((10, _RB, D), lambda r: (0, r, 0)),
            pl.BlockSpec((10, _RB, D), lambda r: (0, r, 0)),
            pl.BlockSpec((D, D), lambda r: (0, 0)),
            pl.BlockSpec((1, D), lambda r: (0, 0)),
            pl.BlockSpec((D, D), lambda r: (0, 0)),
            pl.BlockSpec((D, D), lambda r: (0, 0)),
            pl.BlockSpec((1, D), lambda r: (0, 0)),
            pl.BlockSpec((D, D), lambda r: (0, 0)),
        ],
        out_specs=pl.BlockSpec((10, _RB, D), lambda r: (0, r, 0)),
        out_shape=jax.ShapeDtypeStruct((10, N, D), jnp.float32),
    )(agg, cnt, x, res, wlT_u, bl_u, wrT_u, wlT_v, bl_v, wrT_v)


def _s2s_body(x_ref, wihT_ref, whhT_ref, bih_ref, bhh_ref, o_ref):
    x = x_ref[0]
    wihT = wihT_ref[...]
    whhT = whhT_ref[...]
    bih = bih_ref[...]
    bhh = bhh_ref[...]
    q_star = jnp.zeros((B, 2 * D), jnp.float32)
    h = jnp.zeros((B, D), jnp.float32)
    c = jnp.zeros((B, D), jnp.float32)
    for _ in range(2):
        gates = (jnp.dot(q_star, wihT, preferred_element_type=jnp.float32) + bih
                 + jnp.dot(h, whhT, preferred_element_type=jnp.float32) + bhh)
        ig = jax.nn.sigmoid(gates[:, :D])
        fg = jax.nn.sigmoid(gates[:, D:2 * D])
        gg = jnp.tanh(gates[:, 2 * D:3 * D])
        og = jax.nn.sigmoid(gates[:, 3 * D:])
        c = fg * c + ig * gg
        h = og * jnp.tanh(c)
        e = jnp.sum(x * h[:, None, :], axis=-1)
        m = jnp.max(e, axis=1, keepdims=True)
        ex = jnp.exp(e - m)
        s = jnp.sum(ex, axis=1, keepdims=True)
        a = ex / s
        r = jnp.sum(x * a[:, :, None], axis=1)
        q_star = jnp.concatenate([h, r], axis=1)
    o_ref[0] = q_star


def _s2s_all(x4, wihT, whhT, bih2, bhh2):
    return pl.pallas_call(
        _s2s_body,
        grid=(10,),
        in_specs=[
            pl.BlockSpec((1, B, LEN, D), lambda j: (j, 0, 0, 0)),
            pl.BlockSpec((2 * D, 4 * D), lambda j: (0, 0)),
            pl.BlockSpec((D, 4 * D), lambda j: (0, 0)),
            pl.BlockSpec((1, 4 * D), lambda j: (0, 0)),
            pl.BlockSpec((1, 4 * D), lambda j: (0, 0)),
        ],
        out_specs=pl.BlockSpec((1, B, 2 * D), lambda j: (j, 0, 0)),
        out_shape=jax.ShapeDtypeStruct((10, B, 2 * D), jnp.float32),
    )(x4, wihT, whhT, bih2, bhh2)


def kernel(solute_data_zero, solute_data_one, solute_data_two, solute_data_three, solute_data_four, solvent_data_zero, solvent_data_one, solvent_data_two, solvent_data_three, solvent_data_four, solute_to_embedding, smile_zero, smile_one, smile_two, smile_three, smile_four, solute_adj, solvent_adj_zero, solvent_adj_one, solvent_adj_two, solvent_adj_three, solvent_adj_four, fc1_W, fc1_b, solute_c1_Wl, solute_c1_bl, solute_c1_Wr, solute_c2_Wl, solute_c2_bl, solute_c2_Wr, solvent_c1_Wl, solvent_c1_bl, solvent_c1_Wr, solvent_c2_Wl, solvent_c2_bl, solvent_c2_Wr, gru_Wih_f, gru_bih_f, gru_bhh_f, gru_Wih_b, gru_bih_b, gru_bhh_b, s2s_Wih, s2s_Whh, s2s_bih, s2s_bhh):
    datas = [d.reshape(N, NFEAT) for d in
             (solute_data_zero, solute_data_one, solute_data_two, solute_data_three, solute_data_four,
              solvent_data_zero, solvent_data_one, solvent_data_two, solvent_data_three, solvent_data_four)]
    adjs = tuple(a.reshape(2, E // CHUNK, CHUNK) for a in
                 (solute_adj, solvent_adj_zero, solvent_adj_one, solvent_adj_two,
                  solvent_adj_three, solvent_adj_four))
    zeros64 = jnp.zeros((RPT, D), jnp.float32)
    zeros16 = jnp.zeros((RPT, 16), jnp.float32)
    ones16 = jnp.ones((CHUNK, 16), jnp.float32)

    x0 = _fc1_all(datas, fc1_W.T, fc1_b.reshape(1, D))

    agg1, cnt = _sc_seg_counts(x0, *adjs, zeros64, zeros16, ones16)

    x1 = _sage_all(True, False, agg1, cnt, x0, x0,
                   solute_c1_Wl.T, solute_c1_bl.reshape(1, D), solute_c1_Wr.T,
                   solvent_c1_Wl.T, solvent_c1_bl.reshape(1, D), solvent_c1_Wr.T)

    agg2 = _sc_seg_plain(x1, *adjs, zeros64, zeros16, ones16)

    x2 = _sage_all(False, True, agg2, cnt, x1, x0,
                   solute_c2_Wl.T, solute_c2_bl.reshape(1, D), solute_c2_Wr.T,
                   solvent_c2_Wl.T, solvent_c2_bl.reshape(1, D), solvent_c2_Wr.T)

    x4 = x2.reshape(10, B, LEN, D)
    pooled = _s2s_all(x4, s2s_Wih.T, s2s_Whh.T,
                      s2s_bih.reshape(1, 4 * D), s2s_bhh.reshape(1, 4 * D))

    out = jnp.concatenate([pooled[:5], pooled[5:]], axis=2)
    return out.reshape(5 * B, 4 * D)


# speedup vs baseline: 14.4749x; 1.3227x over previous
import functools

import jax
import jax.numpy as jnp
from jax import lax
from jax.experimental import pallas as pl
from jax.experimental.pallas import tpu as pltpu
from jax.experimental.pallas import tpu_sc as plsc

B = 64
LEN = 256
NFEAT = 128
D = 64
E = 262144
N = B * LEN

NC = 2
NS = 16
EPC = E // NC
EPT = EPC // NS
CHUNK = 256
NCHUNK = EPT // CHUNK
RPT = N // NS

_mesh = plsc.VectorSubcoreMesh(core_axis_name="c", subcore_axis_name="s")
_sc_params = pltpu.CompilerParams(use_tc_tiling_on_sc=False)


def _seg_job(x_hbm, adj_hbm, agg_hbm, j, cid, sid, acc, zrows_hbm, sidxall,
             didxall, rows, sems):
    pltpu.sync_copy(zrows_hbm, acc.at[pl.ds(sid * RPT, RPT), :])
    cbase = (cid * EPC + sid * EPT) // CHUNK
    pltpu.sync_copy(adj_hbm.at[0, pl.ds(cbase, NCHUNK), :], sidxall)
    pltpu.sync_copy(adj_hbm.at[1, pl.ds(cbase, NCHUNK), :], didxall)
    plsc.subcore_barrier()

    def issue(k, b):
        pltpu.async_copy(x_hbm.at[sidxall.at[k]], rows[b], sems[b])

    for b in range(2):
        issue(b, b)

    def outer(i, carry):
        for b in range(2):
            k = 2 * i + b
            pltpu.make_async_copy(x_hbm.at[sidxall.at[k]], rows[b], sems[b]).wait()
            pltpu.sync_copy(rows[b], acc.at[didxall.at[k]], add=True)

            @pl.when(k + 2 < NCHUNK)
            def _():
                issue(k + 2, b)
        return carry

    lax.fori_loop(0, NCHUNK // 2, outer, 0)
    plsc.subcore_barrier()
    for c in range(NC):
        @pl.when(cid == c)
        def _():
            pltpu.sync_copy(acc.at[pl.ds(sid * RPT, RPT), :],
                            agg_hbm.at[c, j, pl.ds(sid * RPT, RPT), :])
    plsc.subcore_barrier()


def _make_sc_seg(nadj):
    @functools.partial(
        pl.kernel,
        out_type=jax.ShapeDtypeStruct((NC, 5, N, D), jnp.float32),
        mesh=_mesh,
        compiler_params=_sc_params,
        scratch_types=[
            pltpu.VMEM_SHARED((N, D), jnp.float32),
            pltpu.VMEM((NCHUNK, CHUNK), jnp.int32),
            pltpu.VMEM((NCHUNK, CHUNK), jnp.int32),
            pltpu.VMEM((CHUNK, D), jnp.float32),
            pltpu.VMEM((CHUNK, D), jnp.float32),
            pltpu.SemaphoreType.DMA,
            pltpu.SemaphoreType.DMA,
        ],
    )
    def sc_seg(xs, *rest):
        adjs = rest[:nadj]
        zeros64 = rest[nadj]
        (agg, acc, sidxall, didxall, rows0, rows1, sem0, sem1) = rest[nadj + 1:]
        rows = (rows0, rows1)
        sems = (sem0, sem1)
        cid = lax.axis_index("c")
        sid = lax.axis_index("s")
        for j in range(5):
            adj = adjs[0] if nadj == 1 else adjs[j]
            _seg_job(xs.at[j], adj, agg, j, cid, sid, acc, zeros64,
                     sidxall, didxall, rows, sems)

    return sc_seg


@functools.partial(
    pl.kernel,
    out_type=(jax.ShapeDtypeStruct((NC, 1, N, 16), jnp.float32),
              jax.ShapeDtypeStruct((NC, 5, N, 16), jnp.float32)),
    mesh=_mesh,
    compiler_params=_sc_params,
    scratch_types=[
        pltpu.VMEM_SHARED((N, 16), jnp.float32),
        pltpu.VMEM((CHUNK, 16), jnp.float32),
        pltpu.VMEM((NCHUNK, CHUNK), jnp.int32),
        pltpu.SemaphoreType.DMA,
    ],
)
def _sc_cnt(a0, a1, a2, a3, a4, a5, zeros16, ones16, cnt_su, cnt_sv,
            cacc, obuf16, didxall, sem):
    cid = lax.axis_index("c")
    sid = lax.axis_index("s")
    pltpu.sync_copy(ones16, obuf16)
    adjs = [a0, a1, a2, a3, a4, a5]
    for a in range(6):
        pltpu.sync_copy(zeros16, cacc.at[pl.ds(sid * RPT, RPT), :])
        cbase = (cid * EPC + sid * EPT) // CHUNK
        pltpu.sync_copy(adjs[a].at[1, pl.ds(cbase, NCHUNK), :], didxall)
        plsc.subcore_barrier()

        def chunk(i, carry):
            pltpu.sync_copy(obuf16, cacc.at[didxall.at[i]], add=True)
            return carry

        lax.fori_loop(0, NCHUNK, chunk, 0)
        plsc.subcore_barrier()
        out = cnt_su if a == 0 else cnt_sv
        slot = 0 if a == 0 else a - 1
        for c in range(NC):
            @pl.when(cid == c)
            def _():
                pltpu.sync_copy(cacc.at[pl.ds(sid * RPT, RPT), :],
                                out.at[c, slot, pl.ds(sid * RPT, RPT), :])
        plsc.subcore_barrier()


_sc_seg_su = _make_sc_seg(1)
_sc_seg_sv = _make_sc_seg(5)


_RB = 512
_NG = N // _RB


def _fc1_body(*refs):
    xs = refs[:10]
    w, b = refs[10], refs[11]
    osu, osv = refs[12], refs[13]
    wv = w[...]
    bv = b[...]
    for j in range(5):
        osu[j] = jnp.dot(xs[j][...], wv, preferred_element_type=jnp.float32) + bv
        osv[j] = jnp.dot(xs[5 + j][...], wv, preferred_element_type=jnp.float32) + bv


def _fc1_all(datas, wT, b2):
    return pl.pallas_call(
        _fc1_body,
        grid=(_NG,),
        in_specs=[pl.BlockSpec((_RB, NFEAT), lambda r: (r, 0))] * 10
        + [pl.BlockSpec((NFEAT, D), lambda r: (0, 0)),
           pl.BlockSpec((1, D), lambda r: (0, 0))],
        out_specs=[pl.BlockSpec((5, _RB, D), lambda r: (0, r, 0))] * 2,
        out_shape=[jax.ShapeDtypeStruct((5, N, D), jnp.float32)] * 2,
    )(*datas, wT, b2)


def _sage1_body(G, agg_ref, cnt_ref, x_ref, wlT_r, bl_r, wrT_r, x1_o, rcnt_o):
    scale = 1.0 / jnp.maximum(cnt_ref[0, :, :, 0:1] + cnt_ref[1, :, :, 0:1], 1.0)
    rcnt_o[...] = scale[:, :, 0]
    wl = wlT_r[...]
    b = bl_r[...]
    wr = wrT_r[...]
    for j in range(5):
        g = 0 if G == 1 else j
        mean = (agg_ref[0, j] + agg_ref[1, j]) * scale[g]
        h = (jnp.dot(mean, wl, preferred_element_type=jnp.float32) + b
             + jnp.dot(x_ref[j], wr, preferred_element_type=jnp.float32))
        x1_o[j] = jnp.maximum(h, 0.0)


def _sage1_all(G, agg, cnt, x, wlT, bl, wrT):
    body = functools.partial(_sage1_body, G)
    return pl.pallas_call(
        body,
        grid=(_NG,),
        in_specs=[
            pl.BlockSpec((NC, 5, _RB, D), lambda r: (0, 0, r, 0)),
            pl.BlockSpec((NC, G, _RB, 16), lambda r: (0, 0, r, 0)),
            pl.BlockSpec((5, _RB, D), lambda r: (0, r, 0)),
            pl.BlockSpec((D, D), lambda r: (0, 0)),
            pl.BlockSpec((1, D), lambda r: (0, 0)),
            pl.BlockSpec((D, D), lambda r: (0, 0)),
        ],
        out_specs=[pl.BlockSpec((5, _RB, D), lambda r: (0, r, 0)),
                   pl.BlockSpec((G, _RB), lambda r: (0, r))],
        out_shape=[jax.ShapeDtypeStruct((5, N, D), jnp.float32),
                   jax.ShapeDtypeStruct((G, N), jnp.float32)],
    )(agg, cnt, x, wlT, bl, wrT)


_GB = 16
_GROWS = _GB * LEN


def _sage2_s2s_body(agg_ref, rcnt_ref, x1_ref, x0_ref, wlT_r, bl_r, wrT_r,
                    wihT_r, whhT_r, bih_r, bhh_r, o_ref):
    r_col = jnp.transpose(rcnt_ref[...].reshape(1, _GROWS))
    mean = (agg_ref[0, 0] + agg_ref[1, 0]) * r_col
    x2 = (jnp.dot(mean, wlT_r[...], preferred_element_type=jnp.float32) + bl_r[...]
          + jnp.dot(x1_ref[0], wrT_r[...], preferred_element_type=jnp.float32)
          + x0_ref[0])
    x = x2.reshape(_GB, LEN, D)
    wihT = wihT_r[...]
    whhT = whhT_r[...]
    bih = bih_r[...]
    bhh = bhh_r[...]
    q_star = jnp.zeros((_GB, 2 * D), jnp.float32)
    h = jnp.zeros((_GB, D), jnp.float32)
    c = jnp.zeros((_GB, D), jnp.float32)
    for _ in range(2):
        gates = (jnp.dot(q_star, wihT, preferred_element_type=jnp.float32) + bih
                 + jnp.dot(h, whhT, preferred_element_type=jnp.float32) + bhh)
        ig = jax.nn.sigmoid(gates[:, :D])
        fg = jax.nn.sigmoid(gates[:, D:2 * D])
        gg = jnp.tanh(gates[:, 2 * D:3 * D])
        og = jax.nn.sigmoid(gates[:, 3 * D:])
        c = fg * c + ig * gg
        h = og * jnp.tanh(c)
        e = jnp.sum(x * h[:, None, :], axis=-1)
        m = jnp.max(e, axis=1, keepdims=True)
        ex = jnp.exp(e - m)
        s = jnp.sum(ex, axis=1, keepdims=True)
        a = ex / s
        rr = jnp.sum(x * a[:, :, None], axis=1)
        q_star = jnp.concatenate([h, rr], axis=1)
    o_ref[0] = q_star


def _sage2_s2s_all(G, agg, rcnt, x1, x0, wlT, bl, wrT, wihT, whhT, bih2, bhh2):
    rmap = (lambda j, g: (0, 0, g)) if G == 1 else (lambda j, g: (j, 0, g))
    rcnt = rcnt.reshape(G, 1, N)
    return pl.pallas_call(
        _sage2_s2s_body,
        grid=(5, B // _GB),
        in_specs=[
            pl.BlockSpec((NC, 1, _GROWS, D), lambda j, g: (0, j, g, 0)),
            pl.BlockSpec((1, 1, _GROWS), rmap),
            pl.BlockSpec((1, _GROWS, D), lambda j, g: (j, g, 0)),
            pl.BlockSpec((1, _GROWS, D), lambda j, g: (j, g, 0)),
            pl.BlockSpec((D, D), lambda j, g: (0, 0)),
            pl.BlockSpec((1, D), lambda j, g: (0, 0)),
            pl.BlockSpec((D, D), lambda j, g: (0, 0)),
            pl.BlockSpec((2 * D, 4 * D), lambda j, g: (0, 0)),
            pl.BlockSpec((D, 4 * D), lambda j, g: (0, 0)),
            pl.BlockSpec((1, 4 * D), lambda j, g: (0, 0)),
            pl.BlockSpec((1, 4 * D), lambda j, g: (0, 0)),
        ],
        out_specs=pl.BlockSpec((1, _GB, 2 * D), lambda j, g: (j, g, 0)),
        out_shape=jax.ShapeDtypeStruct((5, B, 2 * D), jnp.float32),
    )(agg, rcnt, x1, x0, wlT, bl, wrT, wihT, whhT, bih2, bhh2)


def kernel(solute_data_zero, solute_data_one, solute_data_two, solute_data_three, solute_data_four, solvent_data_zero, solvent_data_one, solvent_data_two, solvent_data_three, solvent_data_four, solute_to_embedding, smile_zero, smile_one, smile_two, smile_three, smile_four, solute_adj, solvent_adj_zero, solvent_adj_one, solvent_adj_two, solvent_adj_three, solvent_adj_four, fc1_W, fc1_b, solute_c1_Wl, solute_c1_bl, solute_c1_Wr, solute_c2_Wl, solute_c2_bl, solute_c2_Wr, solvent_c1_Wl, solvent_c1_bl, solvent_c1_Wr, solvent_c2_Wl, solvent_c2_bl, solvent_c2_Wr, gru_Wih_f, gru_bih_f, gru_bhh_f, gru_Wih_b, gru_bih_b, gru_bhh_b, s2s_Wih, s2s_Whh, s2s_bih, s2s_bhh):
    datas = [d.reshape(N, NFEAT) for d in
             (solute_data_zero, solute_data_one, solute_data_two, solute_data_three, solute_data_four,
              solvent_data_zero, solvent_data_one, solvent_data_two, solvent_data_three, solvent_data_four)]
    adjs = tuple(a.reshape(2, E // CHUNK, CHUNK) for a in
                 (solute_adj, solvent_adj_zero, solvent_adj_one, solvent_adj_two,
                  solvent_adj_three, solvent_adj_four))
    zeros64 = jnp.zeros((RPT, D), jnp.float32)
    zeros16 = jnp.zeros((RPT, 16), jnp.float32)
    ones16 = jnp.ones((CHUNK, 16), jnp.float32)

    cnt_su, cnt_sv = _sc_cnt(*adjs, zeros16, ones16)

    x0su, x0sv = _fc1_all(datas, fc1_W.T, fc1_b.reshape(1, D))

    agg1su = _sc_seg_su(x0su, adjs[0], zeros64)
    agg1sv = _sc_seg_sv(x0sv, *adjs[1:], zeros64)
    x1su, rcnt_su = _sage1_all(1, agg1su, cnt_su, x0su,
                               solute_c1_Wl.T, solute_c1_bl.reshape(1, D), solute_c1_Wr.T)
    x1sv, rcnt_sv = _sage1_all(5, agg1sv, cnt_sv, x0sv,
                               solvent_c1_Wl.T, solvent_c1_bl.reshape(1, D), solvent_c1_Wr.T)

    agg2su = _sc_seg_su(x1su, adjs[0], zeros64)
    agg2sv = _sc_seg_sv(x1sv, *adjs[1:], zeros64)
    psu = _sage2_s2s_all(1, agg2su, rcnt_su, x1su, x0su,
                         solute_c2_Wl.T, solute_c2_bl.reshape(1, D), solute_c2_Wr.T,
                         s2s_Wih.T, s2s_Whh.T, s2s_bih.reshape(1, 4 * D),
                         s2s_bhh.reshape(1, 4 * D))
    psv = _sage2_s2s_all(5, agg2sv, rcnt_sv, x1sv, x0sv,
                         solvent_c2_Wl.T, solvent_c2_bl.reshape(1, D), solvent_c2_Wr.T,
                         s2s_Wih.T, s2s_Whh.T, s2s_bih.reshape(1, 4 * D),
                         s2s_bhh.reshape(1, 4 * D))

    out = jnp.concatenate([psu, psv], axis=2)
    return out.reshape(5 * B, 4 * D)


# speedup vs baseline: 14.7667x; 1.0202x over previous
import functools

import jax
import jax.numpy as jnp
from jax import lax
from jax.experimental import pallas as pl
from jax.experimental.pallas import tpu as pltpu
from jax.experimental.pallas import tpu_sc as plsc

B = 64
LEN = 256
NFEAT = 128
D = 64
E = 262144
N = B * LEN

NC = 2
NS = 16
EPC = E // NC
EPT = EPC // NS
CHUNK = 256
NCHUNK = EPT // CHUNK
RPT = N // NS

_mesh = plsc.VectorSubcoreMesh(core_axis_name="c", subcore_axis_name="s")
_sc_params = pltpu.CompilerParams(use_tc_tiling_on_sc=False)


def _writeback(agg_hbm, j, cid, sid, acc):
    for c in range(NC):
        @pl.when(cid == c)
        def _():
            pltpu.sync_copy(acc.at[pl.ds(sid * RPT, RPT), :],
                            agg_hbm.at[c, j, pl.ds(sid * RPT, RPT), :])


def _seg_job(x_hbm, adj_hbm, agg_hbm, prev_j, cid, sid, acc, zrows_hbm,
             sidxall, didxall, rows, sems):
    cbase = (cid * EPC + sid * EPT) // CHUNK
    pltpu.sync_copy(adj_hbm.at[0, pl.ds(cbase, NCHUNK), :], sidxall)
    pltpu.sync_copy(adj_hbm.at[1, pl.ds(cbase, NCHUNK), :], didxall)
    if prev_j is not None:
        _writeback(agg_hbm, prev_j, cid, sid, acc)
    pltpu.sync_copy(zrows_hbm, acc.at[pl.ds(sid * RPT, RPT), :])
    plsc.subcore_barrier()

    def issue(k, b):
        pltpu.async_copy(x_hbm.at[sidxall.at[k]], rows[b], sems[b])

    for b in range(2):
        issue(b, b)

    def outer(i, carry):
        for b in range(2):
            k = 2 * i + b
            pltpu.make_async_copy(x_hbm.at[sidxall.at[k]], rows[b], sems[b]).wait()
            pltpu.sync_copy(rows[b], acc.at[didxall.at[k]], add=True)

            @pl.when(k + 2 < NCHUNK)
            def _():
                issue(k + 2, b)
        return carry

    lax.fori_loop(0, NCHUNK // 2, outer, 0)
    plsc.subcore_barrier()


def _make_sc_seg(nadj):
    @functools.partial(
        pl.kernel,
        out_type=jax.ShapeDtypeStruct((NC, 5, N, D), jnp.float32),
        mesh=_mesh,
        compiler_params=_sc_params,
        scratch_types=[
            pltpu.VMEM_SHARED((N, D), jnp.float32),
            pltpu.VMEM((NCHUNK, CHUNK), jnp.int32),
            pltpu.VMEM((NCHUNK, CHUNK), jnp.int32),
            pltpu.VMEM((CHUNK, D), jnp.float32),
            pltpu.VMEM((CHUNK, D), jnp.float32),
            pltpu.SemaphoreType.DMA,
            pltpu.SemaphoreType.DMA,
        ],
    )
    def sc_seg(xs, *rest):
        adjs = rest[:nadj]
        zeros64 = rest[nadj]
        (agg, acc, sidxall, didxall, rows0, rows1, sem0, sem1) = rest[nadj + 1:]
        rows = (rows0, rows1)
        sems = (sem0, sem1)
        cid = lax.axis_index("c")
        sid = lax.axis_index("s")
        for j in range(5):
            adj = adjs[0] if nadj == 1 else adjs[j]
            _seg_job(xs.at[j], adj, agg, j - 1 if j else None, cid, sid, acc,
                     zeros64, sidxall, didxall, rows, sems)
        _writeback(agg, 4, cid, sid, acc)

    return sc_seg


@functools.partial(
    pl.kernel,
    out_type=(jax.ShapeDtypeStruct((NC, 1, N, 16), jnp.float32),
              jax.ShapeDtypeStruct((NC, 5, N, 16), jnp.float32)),
    mesh=_mesh,
    compiler_params=_sc_params,
    scratch_types=[
        pltpu.VMEM_SHARED((N, 16), jnp.float32),
        pltpu.VMEM((CHUNK, 16), jnp.float32),
        pltpu.VMEM((NCHUNK, CHUNK), jnp.int32),
        pltpu.SemaphoreType.DMA,
    ],
)
def _sc_cnt(a0, a1, a2, a3, a4, a5, zeros16, ones16, cnt_su, cnt_sv,
            cacc, obuf16, didxall, sem):
    cid = lax.axis_index("c")
    sid = lax.axis_index("s")
    pltpu.sync_copy(ones16, obuf16)
    adjs = [a0, a1, a2, a3, a4, a5]
    for a in range(6):
        pltpu.sync_copy(zeros16, cacc.at[pl.ds(sid * RPT, RPT), :])
        cbase = (cid * EPC + sid * EPT) // CHUNK
        pltpu.sync_copy(adjs[a].at[1, pl.ds(cbase, NCHUNK), :], didxall)
        plsc.subcore_barrier()

        def chunk(i, carry):
            pltpu.sync_copy(obuf16, cacc.at[didxall.at[i]], add=True)
            return carry

        lax.fori_loop(0, NCHUNK, chunk, 0)
        plsc.subcore_barrier()
        out = cnt_su if a == 0 else cnt_sv
        slot = 0 if a == 0 else a - 1
        for c in range(NC):
            @pl.when(cid == c)
            def _():
                pltpu.sync_copy(cacc.at[pl.ds(sid * RPT, RPT), :],
                                out.at[c, slot, pl.ds(sid * RPT, RPT), :])
        plsc.subcore_barrier()


_sc_seg_su = _make_sc_seg(1)
_sc_seg_sv = _make_sc_seg(5)


_RB = 512
_NG = N // _RB


def _fc1_body(*refs):
    xs = refs[:10]
    w, b = refs[10], refs[11]
    osu, osv = refs[12], refs[13]
    wv = w[...]
    bv = b[...]
    for j in range(5):
        osu[j] = jnp.dot(xs[j][...], wv, preferred_element_type=jnp.float32) + bv
        osv[j] = jnp.dot(xs[5 + j][...], wv, preferred_element_type=jnp.float32) + bv


def _fc1_all(datas, wT, b2):
    return pl.pallas_call(
        _fc1_body,
        grid=(_NG,),
        in_specs=[pl.BlockSpec((_RB, NFEAT), lambda r: (r, 0))] * 10
        + [pl.BlockSpec((NFEAT, D), lambda r: (0, 0)),
           pl.BlockSpec((1, D), lambda r: (0, 0))],
        out_specs=[pl.BlockSpec((5, _RB, D), lambda r: (0, r, 0))] * 2,
        out_shape=[jax.ShapeDtypeStruct((5, N, D), jnp.float32)] * 2,
    )(*datas, wT, b2)


def _sage1_body(G, agg_ref, cnt_ref, x_ref, wlT_r, bl_r, wrT_r, x1_o, rcnt_o):
    scale = 1.0 / jnp.maximum(cnt_ref[0, :, :, 0:1] + cnt_ref[1, :, :, 0:1], 1.0)
    rcnt_o[...] = scale[:, :, 0]
    wl = wlT_r[...]
    b = bl_r[...]
    wr = wrT_r[...]
    for j in range(5):
        g = 0 if G == 1 else j
        mean = (agg_ref[0, j] + agg_ref[1, j]) * scale[g]
        h = (jnp.dot(mean, wl, preferred_element_type=jnp.float32) + b
             + jnp.dot(x_ref[j], wr, preferred_element_type=jnp.float32))
        x1_o[j] = jnp.maximum(h, 0.0)


def _sage1_all(G, agg, cnt, x, wlT, bl, wrT):
    body = functools.partial(_sage1_body, G)
    return pl.pallas_call(
        body,
        grid=(_NG,),
        in_specs=[
            pl.BlockSpec((NC, 5, _RB, D), lambda r: (0, 0, r, 0)),
            pl.BlockSpec((NC, G, _RB, 16), lambda r: (0, 0, r, 0)),
            pl.BlockSpec((5, _RB, D), lambda r: (0, r, 0)),
            pl.BlockSpec((D, D), lambda r: (0, 0)),
            pl.BlockSpec((1, D), lambda r: (0, 0)),
            pl.BlockSpec((D, D), lambda r: (0, 0)),
        ],
        out_specs=[pl.BlockSpec((5, _RB, D), lambda r: (0, r, 0)),
                   pl.BlockSpec((G, _RB), lambda r: (0, r))],
        out_shape=[jax.ShapeDtypeStruct((5, N, D), jnp.float32),
                   jax.ShapeDtypeStruct((G, N), jnp.float32)],
    )(agg, cnt, x, wlT, bl, wrT)


_GB = 16
_GROWS = _GB * LEN


def _sage2_s2s_body(agg_ref, rcnt_ref, x1_ref, x0_ref, wlT_r, bl_r, wrT_r,
                    wihT_r, whhT_r, bih_r, bhh_r, o_ref):
    r_col = jnp.transpose(rcnt_ref[...].reshape(1, _GROWS))
    mean = (agg_ref[0, 0] + agg_ref[1, 0]) * r_col
    x2 = (jnp.dot(mean, wlT_r[...], preferred_element_type=jnp.float32) + bl_r[...]
          + jnp.dot(x1_ref[0], wrT_r[...], preferred_element_type=jnp.float32)
          + x0_ref[0])
    x = x2.reshape(_GB, LEN, D)
    wihT = wihT_r[...]
    whhT = whhT_r[...]
    bih = bih_r[...]
    bhh = bhh_r[...]
    q_star = jnp.zeros((_GB, 2 * D), jnp.float32)
    h = jnp.zeros((_GB, D), jnp.float32)
    c = jnp.zeros((_GB, D), jnp.float32)
    for _ in range(2):
        gates = (jnp.dot(q_star, wihT, preferred_element_type=jnp.float32) + bih
                 + jnp.dot(h, whhT, preferred_element_type=jnp.float32) + bhh)
        ig = jax.nn.sigmoid(gates[:, :D])
        fg = jax.nn.sigmoid(gates[:, D:2 * D])
        gg = jnp.tanh(gates[:, 2 * D:3 * D])
        og = jax.nn.sigmoid(gates[:, 3 * D:])
        c = fg * c + ig * gg
        h = og * jnp.tanh(c)
        e = jnp.sum(x * h[:, None, :], axis=-1)
        m = jnp.max(e, axis=1, keepdims=True)
        ex = jnp.exp(e - m)
        s = jnp.sum(ex, axis=1, keepdims=True)
        a = ex / s
        rr = jnp.sum(x * a[:, :, None], axis=1)
        q_star = jnp.concatenate([h, rr], axis=1)
    o_ref[0] = q_star


def _sage2_s2s_all(G, agg, rcnt, x1, x0, wlT, bl, wrT, wihT, whhT, bih2, bhh2):
    rmap = (lambda j, g: (0, 0, g)) if G == 1 else (lambda j, g: (j, 0, g))
    rcnt = rcnt.reshape(G, 1, N)
    return pl.pallas_call(
        _sage2_s2s_body,
        grid=(5, B // _GB),
        in_specs=[
            pl.BlockSpec((NC, 1, _GROWS, D), lambda j, g: (0, j, g, 0)),
            pl.BlockSpec((1, 1, _GROWS), rmap),
            pl.BlockSpec((1, _GROWS, D), lambda j, g: (j, g, 0)),
            pl.BlockSpec((1, _GROWS, D), lambda j, g: (j, g, 0)),
            pl.BlockSpec((D, D), lambda j, g: (0, 0)),
            pl.BlockSpec((1, D), lambda j, g: (0, 0)),
            pl.BlockSpec((D, D), lambda j, g: (0, 0)),
            pl.BlockSpec((2 * D, 4 * D), lambda j, g: (0, 0)),
            pl.BlockSpec((D, 4 * D), lambda j, g: (0, 0)),
            pl.BlockSpec((1, 4 * D), lambda j, g: (0, 0)),
            pl.BlockSpec((1, 4 * D), lambda j, g: (0, 0)),
        ],
        out_specs=pl.BlockSpec((1, _GB, 2 * D), lambda j, g: (j, g, 0)),
        out_shape=jax.ShapeDtypeStruct((5, B, 2 * D), jnp.float32),
    )(agg, rcnt, x1, x0, wlT, bl, wrT, wihT, whhT, bih2, bhh2)


def kernel(solute_data_zero, solute_data_one, solute_data_two, solute_data_three, solute_data_four, solvent_data_zero, solvent_data_one, solvent_data_two, solvent_data_three, solvent_data_four, solute_to_embedding, smile_zero, smile_one, smile_two, smile_three, smile_four, solute_adj, solvent_adj_zero, solvent_adj_one, solvent_adj_two, solvent_adj_three, solvent_adj_four, fc1_W, fc1_b, solute_c1_Wl, solute_c1_bl, solute_c1_Wr, solute_c2_Wl, solute_c2_bl, solute_c2_Wr, solvent_c1_Wl, solvent_c1_bl, solvent_c1_Wr, solvent_c2_Wl, solvent_c2_bl, solvent_c2_Wr, gru_Wih_f, gru_bih_f, gru_bhh_f, gru_Wih_b, gru_bih_b, gru_bhh_b, s2s_Wih, s2s_Whh, s2s_bih, s2s_bhh):
    datas = [d.reshape(N, NFEAT) for d in
             (solute_data_zero, solute_data_one, solute_data_two, solute_data_three, solute_data_four,
              solvent_data_zero, solvent_data_one, solvent_data_two, solvent_data_three, solvent_data_four)]
    adjs = tuple(a.reshape(2, E // CHUNK, CHUNK) for a in
                 (solute_adj, solvent_adj_zero, solvent_adj_one, solvent_adj_two,
                  solvent_adj_three, solvent_adj_four))
    zeros64 = jnp.zeros((RPT, D), jnp.float32)
    zeros16 = jnp.zeros((RPT, 16), jnp.float32)
    ones16 = jnp.ones((CHUNK, 16), jnp.float32)

    cnt_su, cnt_sv = _sc_cnt(*adjs, zeros16, ones16)

    x0su, x0sv = _fc1_all(datas, fc1_W.T, fc1_b.reshape(1, D))

    agg1su = _sc_seg_su(x0su, adjs[0], zeros64)
    agg1sv = _sc_seg_sv(x0sv, *adjs[1:], zeros64)
    x1su, rcnt_su = _sage1_all(1, agg1su, cnt_su, x0su,
                               solute_c1_Wl.T, solute_c1_bl.reshape(1, D), solute_c1_Wr.T)
    x1sv, rcnt_sv = _sage1_all(5, agg1sv, cnt_sv, x0sv,
                               solvent_c1_Wl.T, solvent_c1_bl.reshape(1, D), solvent_c1_Wr.T)

    agg2su = _sc_seg_su(x1su, adjs[0], zeros64)
    agg2sv = _sc_seg_sv(x1sv, *adjs[1:], zeros64)
    psu = _sage2_s2s_all(1, agg2su, rcnt_su, x1su, x0su,
                         solute_c2_Wl.T, solute_c2_bl.reshape(1, D), solute_c2_Wr.T,
                         s2s_Wih.T, s2s_Whh.T, s2s_bih.reshape(1, 4 * D),
                         s2s_bhh.reshape(1, 4 * D))
    psv = _sage2_s2s_all(5, agg2sv, rcnt_sv, x1sv, x0sv,
                         solvent_c2_Wl.T, solvent_c2_bl.reshape(1, D), solvent_c2_Wr.T,
                         s2s_Wih.T, s2s_Whh.T, s2s_bih.reshape(1, 4 * D),
                         s2s_bhh.reshape(1, 4 * D))

    out = jnp.concatenate([psu, psv], axis=2)
    return out.reshape(5 * B, 4 * D)
